# baseline (speedup 1.0000x reference)
import functools
import math

import jax
import jax.numpy as jnp
from jax import lax
from jax.experimental import pallas as pl
from jax.experimental.pallas import tpu as pltpu
from jax.experimental.pallas import tpu_sc as plsc

VOCAB = 1000000
D_MODEL = 64
B = 4096
L = 200
N_TOK = B * L
SCALE = math.sqrt(D_MODEL)

NC = 2
NS = 16
NW = NC * NS
PER_W = N_TOK // NW
CHUNK = 512
NCHUNK = PER_W // CHUNK

_mesh = plsc.VectorSubcoreMesh(core_axis_name="c", subcore_axis_name="s")


@functools.partial(
    pl.kernel,
    mesh=_mesh,
    compiler_params=pltpu.CompilerParams(use_tc_tiling_on_sc=False),
    out_type=jax.ShapeDtypeStruct((N_TOK, D_MODEL), jnp.float32),
    scratch_types=[
        pltpu.VMEM((PER_W,), jnp.int32),
        pltpu.VMEM((CHUNK, D_MODEL), jnp.float32),
        pltpu.SemaphoreType.DMA,
    ],
)
def _embed(idx_hbm, table_hbm, out_hbm, idx_v, buf, sem):
    wid = lax.axis_index("s") * NC + lax.axis_index("c")
    base = wid * PER_W
    pltpu.sync_copy(idx_hbm.at[pl.ds(base, PER_W)], idx_v)

    def step(g, carry):
        off = pl.multiple_of(g * CHUNK, CHUNK)
        pltpu.async_copy(
            table_hbm.at[idx_v.at[pl.ds(off, CHUNK)]], buf, sem
        ).wait()

        def scale_row(i, c):
            for j in range(D_MODEL // 16):
                sl = pl.ds(j * 16, 16)
                buf[i, sl] = buf[i, sl] * SCALE
            return c

        lax.fori_loop(0, CHUNK, scale_row, 0)
        pltpu.sync_copy(buf, out_hbm.at[pl.ds(base + off, CHUNK)])
        return carry

    lax.fori_loop(0, NCHUNK, step, 0)


def kernel(x, W):
    idx = x.reshape(-1).astype(jnp.int32)
    out = _embed(idx, W)
    return out.reshape(B, L, D_MODEL)


# speedup vs baseline: 1.1163x; 1.1163x over previous
import functools
import math

import jax
import jax.numpy as jnp
from jax import lax
from jax.experimental import pallas as pl
from jax.experimental.pallas import tpu as pltpu
from jax.experimental.pallas import tpu_sc as plsc

VOCAB = 1000000
D_MODEL = 64
B = 4096
L = 200
N_TOK = B * L
SCALE = math.sqrt(D_MODEL)

NC = 2
NS = 16
NW = NC * NS
PER_W = N_TOK // NW
CHUNK = 400
NCHUNK = PER_W // CHUNK
NBUF = 4

_mesh = plsc.VectorSubcoreMesh(core_axis_name="c", subcore_axis_name="s")


@functools.partial(
    pl.kernel,
    mesh=_mesh,
    compiler_params=pltpu.CompilerParams(use_tc_tiling_on_sc=False),
    out_type=jax.ShapeDtypeStruct((N_TOK, D_MODEL), jnp.float32),
    scratch_types=[
        pltpu.VMEM((PER_W,), jnp.int32),
        pltpu.VMEM((NBUF, CHUNK, D_MODEL), jnp.float32),
    ]
    + [pltpu.SemaphoreType.DMA] * (2 * NBUF),
)
def _embed(idx_hbm, table_hbm, out_hbm, idx_v, bufs, *sems):
    gsem = sems[:NBUF]
    osem = sems[NBUF:]
    wid = lax.axis_index("s") * NC + lax.axis_index("c")
    base = wid * PER_W
    pltpu.sync_copy(idx_hbm.at[pl.ds(base, PER_W)], idx_v)

    def fire_gather(k, b):
        off = pl.multiple_of(k * CHUNK, CHUNK)
        pltpu.async_copy(
            table_hbm.at[idx_v.at[pl.ds(off, CHUNK)]], bufs.at[b], gsem[b]
        )

    def wait_gather(k, b):
        off = pl.multiple_of(k * CHUNK, CHUNK)
        pltpu.make_async_copy(
            table_hbm.at[idx_v.at[pl.ds(off, CHUNK)]], bufs.at[b], gsem[b]
        ).wait()

    def fire_out(k, b):
        off = pl.multiple_of(k * CHUNK, CHUNK)
        pltpu.async_copy(bufs.at[b], out_hbm.at[pl.ds(base + off, CHUNK)], osem[b])

    def wait_out(k, b):
        off = pl.multiple_of(k * CHUNK, CHUNK)
        pltpu.make_async_copy(
            bufs.at[b], out_hbm.at[pl.ds(base + off, CHUNK)], osem[b]
        ).wait()

    def scale(b):
        def scale_rows(i, c):
            for r in range(2):
                for j in range(D_MODEL // 16):
                    sl = pl.ds(j * 16, 16)
                    bufs[b, 2 * i + r, sl] = bufs[b, 2 * i + r, sl] * SCALE
            return c

        lax.fori_loop(0, CHUNK // 2, scale_rows, 0)

    fire_gather(0, 0)
    fire_gather(1, 1)

    def step(g, carry):
        for b in range(NBUF):
            k = g * NBUF + b
            j = k + 2
            b2 = (b + 2) % NBUF
            wait_gather(k, b)
            scale(b)
            fire_out(k, b)
            if b < 2:
                @pl.when(g >= 1)
                def _():
                    wait_out(j - NBUF, b2)
                    fire_gather(j, b2)

                @pl.when(g < 1)
                def _():
                    fire_gather(j, b2)
            else:
                wait_out(j - NBUF, b2)

                @pl.when(j < NCHUNK)
                def _():
                    fire_gather(j, b2)
        return carry

    lax.fori_loop(0, NCHUNK // NBUF, step, 0)

    wait_out(NCHUNK - 2, (NCHUNK - 2) % NBUF)
    wait_out(NCHUNK - 1, (NCHUNK - 1) % NBUF)


def kernel(x, W):
    idx = x.reshape(-1).astype(jnp.int32)
    out = _embed(idx, W)
    return out.reshape(B, L, D_MODEL)
